# Initial kernel scaffold; baseline (speedup 1.0000x reference)
import jax
import jax.numpy as jnp
from jax.experimental import pallas as pl


def kernel(x, edge_index, edge_attr, W_l1, b_l1, W_r1, b_r1, W_e1, att1, bias1, W_l2, b_l2, W_r2, b_r2, W_e2, att2, bias2, Wd1, bd1, Wd2, bd2):
    raise NotImplementedError("write your pallas kernel here")


# speedup vs baseline: 42.0588x; 42.0588x over previous
import functools

import jax
import jax.numpy as jnp
from jax import lax
from jax.experimental import pallas as pl
from jax.experimental.pallas import tpu as pltpu
from jax.experimental.pallas import tpu_sc as plsc

N = 10000
E = 640000
NODE_DIM = 128
EDGE_DIM = 16
EMB = 16
HEADS = 2
HC = EMB * HEADS
HID = 32
OUT = 2

NC = 2
NS = 16
NW = NC * NS

ROW = 128
RPS = 8
SUP = ROW * RPS
NSUP = E // SUP
KMAX = -(-NSUP // NW)
QROWS = 2
QE = ROW * QROWS
NPAD = 10240
NPS = NPAD // NS

AW = 40
SW = 32

_sc_mesh = plsc.VectorSubcoreMesh(
    core_axis_name="c", subcore_axis_name="s", num_cores=NC, num_subcores=NS)
_sc_params = pltpu.CompilerParams(use_tc_tiling_on_sc=False)


def _worker_id():
  return lax.axis_index("s") * NC + lax.axis_index("c")


def _sc_stage0_body(dst3, ea_aug, zeros_sw,
                    stat_out,
                    idx_v, rows_v, stat_sh):
  cid = lax.axis_index("c")
  sid = lax.axis_index("s")
  wid = _worker_id()

  @pl.when(sid == 0)
  def _():
    pltpu.sync_copy(zeros_sw, stat_sh)

  plsc.subcore_barrier()

  def body(k, _):
    sblk = wid + k * NW

    @pl.when(sblk < NSUP)
    def _():
      e0 = sblk * SUP
      pltpu.sync_copy(dst3.at[sblk], idx_v)
      for q in range(RPS // QROWS):
        pltpu.sync_copy(ea_aug.at[pl.ds(e0 + q * QE, QE)], rows_v)
        for j in range(QROWS):
          pltpu.sync_copy(rows_v.at[pl.ds(j * ROW, ROW)],
                          stat_sh.at[idx_v.at[q * QROWS + j]], add=True)
    return ()

  lax.fori_loop(0, KMAX, body, ())
  plsc.subcore_barrier()

  n0 = sid * NPS
  pltpu.sync_copy(stat_sh.at[pl.ds(n0, NPS)], stat_out.at[cid, pl.ds(n0, NPS)])


_sc_stage0 = pl.kernel(
    _sc_stage0_body,
    compiler_params=_sc_params,
    out_type=jax.ShapeDtypeStruct((NC, NPAD, SW), jnp.float32),
    mesh=_sc_mesh,
    scratch_types=[
        pltpu.VMEM((RPS, ROW), jnp.int32),
        pltpu.VMEM((QE, SW), jnp.float32),
        pltpu.VMEM_SHARED((NPAD, SW), jnp.float32),
    ],
)


def _sc_gather_body(src3, dst3, xl, xr,
                    gl, gr,
                    idxs, idxd, bufl, bufr, sem):
  wid = _worker_id()

  def body(k, _):
    sblk = wid + k * NW

    @pl.when(sblk < NSUP)
    def _():
      e0 = sblk * SUP
      pltpu.sync_copy(src3.at[sblk], idxs)
      pltpu.sync_copy(dst3.at[sblk], idxd)
      for q in range(RPS // QROWS):
        cps = []
        for j in range(QROWS):
          cps.append(pltpu.async_copy(
              xl.at[idxs.at[q * QROWS + j]],
              bufl.at[pl.ds(j * ROW, ROW)], sem))
          cps.append(pltpu.async_copy(
              xr.at[idxd.at[q * QROWS + j]],
              bufr.at[pl.ds(j * ROW, ROW)], sem))
        for cp in cps:
          cp.wait()
        pltpu.sync_copy(bufl, gl.at[pl.ds(e0 + q * QE, QE)])
        pltpu.sync_copy(bufr, gr.at[pl.ds(e0 + q * QE, QE)])
    return ()

  lax.fori_loop(0, KMAX, body, ())


_sc_gather = pl.kernel(
    _sc_gather_body,
    compiler_params=_sc_params,
    out_type=(jax.ShapeDtypeStruct((E, HC), jnp.float32),
              jax.ShapeDtypeStruct((E, HC), jnp.float32)),
    mesh=_sc_mesh,
    scratch_types=[
        pltpu.VMEM((RPS, ROW), jnp.int32),
        pltpu.VMEM((RPS, ROW), jnp.int32),
        pltpu.VMEM((QE, HC), jnp.float32),
        pltpu.VMEM((QE, HC), jnp.float32),
        pltpu.SemaphoreType.DMA,
    ],
)


def _sc_scatter_body(dst3, p, zeros_aw,
                     acc,
                     idx_v, pb, acc_sh):
  cid = lax.axis_index("c")
  sid = lax.axis_index("s")
  wid = _worker_id()

  @pl.when(sid == 0)
  def _():
    pltpu.sync_copy(zeros_aw, acc_sh)

  plsc.subcore_barrier()

  def body(k, _):
    sblk = wid + k * NW

    @pl.when(sblk < NSUP)
    def _():
      e0 = sblk * SUP
      pltpu.sync_copy(dst3.at[sblk], idx_v)
      for q in range(RPS // QROWS):
        pltpu.sync_copy(p.at[pl.ds(e0 + q * QE, QE)], pb)
        for j in range(QROWS):
          pltpu.sync_copy(pb.at[pl.ds(j * ROW, ROW)],
                          acc_sh.at[idx_v.at[q * QROWS + j]], add=True)
    return ()

  lax.fori_loop(0, KMAX, body, ())
  plsc.subcore_barrier()

  n0 = sid * NPS
  pltpu.sync_copy(acc_sh.at[pl.ds(n0, NPS)], acc.at[cid, pl.ds(n0, NPS)])


_sc_scatter = pl.kernel(
    _sc_scatter_body,
    compiler_params=_sc_params,
    out_type=jax.ShapeDtypeStruct((NC, NPAD, AW), jnp.float32),
    mesh=_sc_mesh,
    scratch_types=[
        pltpu.VMEM((RPS, ROW), jnp.int32),
        pltpu.VMEM((QE, AW), jnp.float32),
        pltpu.VMEM_SHARED((NPAD, AW), jnp.float32),
    ],
)


BE = 10000
BN = 2000


def _edge_e_body(ea_ref, we1_ref, we2_ref, e1_ref, e2_ref, eaaug_ref):
  ea = ea_ref[...]
  e1_ref[...] = jnp.dot(ea, we1_ref[...], preferred_element_type=jnp.float32)
  e2_ref[...] = jnp.dot(ea, we2_ref[...], preferred_element_type=jnp.float32)
  eaaug_ref[...] = jnp.concatenate(
      [ea, jnp.ones((BE, 1), jnp.float32),
       jnp.zeros((BE, SW - EDGE_DIM - 1), jnp.float32)], axis=1)


def _tc_edge_e(ea, we1, we2):
  return pl.pallas_call(
      _edge_e_body,
      grid=(E // BE,),
      in_specs=[
          pl.BlockSpec((BE, EDGE_DIM), lambda i: (i, 0)),
          pl.BlockSpec((EDGE_DIM, HC), lambda i: (0, 0)),
          pl.BlockSpec((EDGE_DIM, HC), lambda i: (0, 0)),
      ],
      out_specs=[
          pl.BlockSpec((BE, HC), lambda i: (i, 0)),
          pl.BlockSpec((BE, HC), lambda i: (i, 0)),
          pl.BlockSpec((BE, SW), lambda i: (i, 0)),
      ],
      out_shape=[
          jax.ShapeDtypeStruct((E, HC), jnp.float32),
          jax.ShapeDtypeStruct((E, HC), jnp.float32),
          jax.ShapeDtypeStruct((E, SW), jnp.float32),
      ],
  )(ea, we1, we2)


def _lrelu(x):
  return jnp.maximum(x, 0.2 * x)


def _attention_rows(xl, m, attf, nrows):
  t = _lrelu(m) * attf
  a0 = jnp.sum(t[:, :EMB], axis=1, keepdims=True)
  a1 = jnp.sum(t[:, EMB:], axis=1, keepdims=True)
  ex0 = jnp.exp(a0)
  ex1 = jnp.exp(a1)
  return jnp.concatenate(
      [xl[:, :EMB] * ex0, xl[:, EMB:] * ex1, ex0, ex1,
       jnp.zeros((nrows, AW - HC - HEADS), jnp.float32)], axis=1)


def _edgewise_body(attf_ref, gl_ref, gr_ref, ee_ref, p_ref):
  gl = gl_ref[...]
  m = gl + gr_ref[...] + ee_ref[...]
  p_ref[...] = _attention_rows(gl, m, attf_ref[...], BE)


def _tc_edgewise(attf, gl, gr, ee):
  return pl.pallas_call(
      _edgewise_body,
      grid=(E // BE,),
      in_specs=[
          pl.BlockSpec((1, HC), lambda i: (0, 0)),
          pl.BlockSpec((BE, HC), lambda i: (i, 0)),
          pl.BlockSpec((BE, HC), lambda i: (i, 0)),
          pl.BlockSpec((BE, HC), lambda i: (i, 0)),
      ],
      out_specs=pl.BlockSpec((BE, AW), lambda i: (i, 0)),
      out_shape=jax.ShapeDtypeStruct((E, AW), jnp.float32),
  )(attf, gl, gr, ee)


def _node1_body(x_ref, stat_ref, wl_ref, bl_ref, wr_ref, br_ref,
                we1_ref, we2_ref, attf_ref,
                xl_ref, xr_ref, e2l_ref, self1_ref):
  stat = stat_ref[0] + stat_ref[1]
  sums = stat[:, :EDGE_DIM]
  deg = stat[:, EDGE_DIM:EDGE_DIM + 1]
  la = sums / jnp.maximum(deg, 1.0)
  x = x_ref[...]
  xl = jnp.dot(x, wl_ref[...], preferred_element_type=jnp.float32) + bl_ref[...]
  xr = jnp.dot(x, wr_ref[...], preferred_element_type=jnp.float32) + br_ref[...]
  e1l = jnp.dot(la, we1_ref[...], preferred_element_type=jnp.float32)
  e2l_ref[...] = jnp.dot(la, we2_ref[...], preferred_element_type=jnp.float32)
  xl_ref[...] = xl
  xr_ref[...] = xr
  self1_ref[...] = _attention_rows(xl, xl + xr + e1l, attf_ref[...], BN)


def _tc_node1(x, stat_p, wl, bl, wr, br, we1, we2, attf):
  return pl.pallas_call(
      _node1_body,
      grid=(N // BN,),
      in_specs=[
          pl.BlockSpec((BN, NODE_DIM), lambda i: (i, 0)),
          pl.BlockSpec((NC, BN, SW), lambda i: (0, i, 0)),
          pl.BlockSpec((NODE_DIM, HC), lambda i: (0, 0)),
          pl.BlockSpec((1, HC), lambda i: (0, 0)),
          pl.BlockSpec((NODE_DIM, HC), lambda i: (0, 0)),
          pl.BlockSpec((1, HC), lambda i: (0, 0)),
          pl.BlockSpec((EDGE_DIM, HC), lambda i: (0, 0)),
          pl.BlockSpec((EDGE_DIM, HC), lambda i: (0, 0)),
          pl.BlockSpec((1, HC), lambda i: (0, 0)),
      ],
      out_specs=[
          pl.BlockSpec((BN, HC), lambda i: (i, 0)),
          pl.BlockSpec((BN, HC), lambda i: (i, 0)),
          pl.BlockSpec((BN, HC), lambda i: (i, 0)),
          pl.BlockSpec((BN, AW), lambda i: (i, 0)),
      ],
      out_shape=[
          jax.ShapeDtypeStruct((N, HC), jnp.float32),
          jax.ShapeDtypeStruct((N, HC), jnp.float32),
          jax.ShapeDtypeStruct((N, HC), jnp.float32),
          jax.ShapeDtypeStruct((N, AW), jnp.float32),
      ],
  )(x, stat_p, wl, bl, wr, br, we1, we2, attf)


def _combine(acc_ref, self_ref, bias):
  num = acc_ref[0, :, :HC] + acc_ref[1, :, :HC] + self_ref[:, :HC]
  d0 = acc_ref[0, :, HC:HC + 1] + acc_ref[1, :, HC:HC + 1] \
      + self_ref[:, HC:HC + 1] + 1e-16
  d1 = acc_ref[0, :, HC + 1:HC + 2] + acc_ref[1, :, HC + 1:HC + 2] \
      + self_ref[:, HC + 1:HC + 2] + 1e-16
  return jnp.concatenate([num[:, :EMB] / d0, num[:, EMB:] / d1], axis=1) + bias


def _node2_body(acc_ref, self1_ref, e2l_ref, bias1_ref,
                wl_ref, bl_ref, wr_ref, br_ref, attf_ref,
                xl_ref, xr_ref, self2_ref):
  h = _combine(acc_ref, self1_ref, bias1_ref[...])
  xl = jnp.dot(h, wl_ref[...], preferred_element_type=jnp.float32) + bl_ref[...]
  xr = jnp.dot(h, wr_ref[...], preferred_element_type=jnp.float32) + br_ref[...]
  xl_ref[...] = xl
  xr_ref[...] = xr
  self2_ref[...] = _attention_rows(xl, xl + xr + e2l_ref[...], attf_ref[...], BN)


def _tc_node2(acc1, self1, e2l, bias1, wl, bl, wr, br, attf):
  return pl.pallas_call(
      _node2_body,
      grid=(N // BN,),
      in_specs=[
          pl.BlockSpec((NC, BN, AW), lambda i: (0, i, 0)),
          pl.BlockSpec((BN, AW), lambda i: (i, 0)),
          pl.BlockSpec((BN, HC), lambda i: (i, 0)),
          pl.BlockSpec((1, HC), lambda i: (0, 0)),
          pl.BlockSpec((HC, HC), lambda i: (0, 0)),
          pl.BlockSpec((1, HC), lambda i: (0, 0)),
          pl.BlockSpec((HC, HC), lambda i: (0, 0)),
          pl.BlockSpec((1, HC), lambda i: (0, 0)),
          pl.BlockSpec((1, HC), lambda i: (0, 0)),
      ],
      out_specs=[
          pl.BlockSpec((BN, HC), lambda i: (i, 0)),
          pl.BlockSpec((BN, HC), lambda i: (i, 0)),
          pl.BlockSpec((BN, AW), lambda i: (i, 0)),
      ],
      out_shape=[
          jax.ShapeDtypeStruct((N, HC), jnp.float32),
          jax.ShapeDtypeStruct((N, HC), jnp.float32),
          jax.ShapeDtypeStruct((N, AW), jnp.float32),
      ],
  )(acc1, self1, e2l, bias1, wl, bl, wr, br, attf)


def _final_body(acc_ref, self2_ref, bias2_ref, wd1_ref, bd1_ref,
                wd2_ref, bd2_ref, q_ref):
  h = _combine(acc_ref, self2_ref, bias2_ref[...])
  t = jnp.maximum(
      jnp.dot(h, wd1_ref[...], preferred_element_type=jnp.float32)
      + bd1_ref[...], 0.0)
  q_ref[...] = jnp.dot(
      t, wd2_ref[...], preferred_element_type=jnp.float32) + bd2_ref[...]


def _tc_final(acc2, self2, bias2, wd1, bd1, wd2, bd2):
  return pl.pallas_call(
      _final_body,
      grid=(N // BN,),
      in_specs=[
          pl.BlockSpec((NC, BN, AW), lambda i: (0, i, 0)),
          pl.BlockSpec((BN, AW), lambda i: (i, 0)),
          pl.BlockSpec((1, HC), lambda i: (0, 0)),
          pl.BlockSpec((HC, HID), lambda i: (0, 0)),
          pl.BlockSpec((1, HID), lambda i: (0, 0)),
          pl.BlockSpec((HID, OUT), lambda i: (0, 0)),
          pl.BlockSpec((1, OUT), lambda i: (0, 0)),
      ],
      out_specs=pl.BlockSpec((BN, OUT), lambda i: (i, 0)),
      out_shape=jax.ShapeDtypeStruct((N, OUT), jnp.float32),
  )(acc2, self2, bias2, wd1, bd1, wd2, bd2)


def kernel(x, edge_index, edge_attr, W_l1, b_l1, W_r1, b_r1, W_e1, att1,
           bias1, W_l2, b_l2, W_r2, b_r2, W_e2, att2, bias2, Wd1, bd1,
           Wd2, bd2):
  src3 = edge_index[0].reshape(NSUP, RPS, ROW)
  dst3 = edge_index[1].reshape(NSUP, RPS, ROW)
  attf1 = att1.reshape(1, HC)
  attf2 = att2.reshape(1, HC)
  zeros_sw = jnp.zeros((NPAD, SW), jnp.float32)
  zeros_aw = jnp.zeros((NPAD, AW), jnp.float32)

  e1, e2, ea_aug = _tc_edge_e(edge_attr, W_e1, W_e2)
  stat_p = _sc_stage0(dst3, ea_aug, zeros_sw)
  xl1, xr1, e2l, self1 = _tc_node1(
      x, stat_p, W_l1, b_l1.reshape(1, HC), W_r1, b_r1.reshape(1, HC),
      W_e1, W_e2, attf1)
  gl1, gr1 = _sc_gather(src3, dst3, xl1, xr1)
  p1 = _tc_edgewise(attf1, gl1, gr1, e1)
  acc1 = _sc_scatter(dst3, p1, zeros_aw)
  xl2, xr2, self2 = _tc_node2(
      acc1, self1, e2l, bias1.reshape(1, HC), W_l2, b_l2.reshape(1, HC),
      W_r2, b_r2.reshape(1, HC), attf2)
  gl2, gr2 = _sc_gather(src3, dst3, xl2, xr2)
  p2 = _tc_edgewise(attf2, gl2, gr2, e2)
  acc2 = _sc_scatter(dst3, p2, zeros_aw)
  q = _tc_final(acc2, self2, bias2.reshape(1, HC), Wd1, bd1.reshape(1, HID),
                Wd2, bd2.reshape(1, OUT))
  return q


# speedup vs baseline: 63.2693x; 1.5043x over previous
import jax
import jax.numpy as jnp
from jax import lax
from jax.experimental import pallas as pl
from jax.experimental.pallas import tpu as pltpu
from jax.experimental.pallas import tpu_sc as plsc

N = 10000
E = 640000
NODE_DIM = 128
EDGE_DIM = 16
EMB = 16
HEADS = 2
HC = EMB * HEADS
HID = 32
OUT = 2

NC = 2
NS = 16
NW = NC * NS

ROW = 128
RPS = 8
SUP = ROW * RPS
NSUP = E // SUP
KMAX = -(-NSUP // NW)
QROWS = 2
QE = ROW * QROWS
NPAD = 10240
NPS = NPAD // NS

_sc_mesh = plsc.VectorSubcoreMesh(
    core_axis_name="c", subcore_axis_name="s", num_cores=NC, num_subcores=NS)
_sc_params = pltpu.CompilerParams(use_tc_tiling_on_sc=False)


def _worker_id():
  return lax.axis_index("s") * NC + lax.axis_index("c")


def _sc_gather_body(src3, dst3, xl, xr,
                    gl, gr,
                    idxs, idxd, bufl, bufr, sem):
  wid = _worker_id()

  def body(k, _):
    sblk = wid + k * NW

    @pl.when(sblk < NSUP)
    def _():
      e0 = sblk * SUP
      pltpu.sync_copy(src3.at[sblk], idxs)
      pltpu.sync_copy(dst3.at[sblk], idxd)
      for q in range(RPS // QROWS):
        cps = []
        for j in range(QROWS):
          cps.append(pltpu.async_copy(
              xl.at[idxs.at[q * QROWS + j]],
              bufl.at[pl.ds(j * ROW, ROW)], sem))
          cps.append(pltpu.async_copy(
              xr.at[idxd.at[q * QROWS + j]],
              bufr.at[pl.ds(j * ROW, ROW)], sem))
        for cp in cps:
          cp.wait()
        pltpu.sync_copy(bufl, gl.at[pl.ds(e0 + q * QE, QE)])
        pltpu.sync_copy(bufr, gr.at[pl.ds(e0 + q * QE, QE)])
    return ()

  lax.fori_loop(0, KMAX, body, ())


_sc_gather = pl.kernel(
    _sc_gather_body,
    compiler_params=_sc_params,
    out_type=(jax.ShapeDtypeStruct((E, HC), jnp.float32),
              jax.ShapeDtypeStruct((E, HC), jnp.float32)),
    mesh=_sc_mesh,
    scratch_types=[
        pltpu.VMEM((RPS, ROW), jnp.int32),
        pltpu.VMEM((RPS, ROW), jnp.int32),
        pltpu.VMEM((QE, HC), jnp.float32),
        pltpu.VMEM((QE, HC), jnp.float32),
        pltpu.SemaphoreType.DMA,
    ],
)


def _sc_scatter_body(dst3, pa, pb, pe, zeros32,
                     acca, accb, accs,
                     idx_v, pab, pbb, peb, acca_sh, accb_sh, accs_sh):
  cid = lax.axis_index("c")
  sid = lax.axis_index("s")
  wid = _worker_id()

  @pl.when(sid == 0)
  def _():
    pltpu.sync_copy(zeros32, acca_sh)
    pltpu.sync_copy(zeros32, accb_sh)
    pltpu.sync_copy(zeros32, accs_sh)

  plsc.subcore_barrier()

  def body(k, _):
    sblk = wid + k * NW

    @pl.when(sblk < NSUP)
    def _():
      e0 = sblk * SUP
      pltpu.sync_copy(dst3.at[sblk], idx_v)
      for q in range(RPS // QROWS):
        pltpu.sync_copy(pa.at[pl.ds(e0 + q * QE, QE)], pab)
        pltpu.sync_copy(pb.at[pl.ds(e0 + q * QE, QE)], pbb)
        pltpu.sync_copy(pe.at[pl.ds(e0 + q * QE, QE)], peb)
        for j in range(QROWS):
          r = idx_v.at[q * QROWS + j]
          sl = pl.ds(j * ROW, ROW)
          pltpu.sync_copy(pab.at[sl], acca_sh.at[r], add=True)
          pltpu.sync_copy(pbb.at[sl], accb_sh.at[r], add=True)
          pltpu.sync_copy(peb.at[sl], accs_sh.at[r], add=True)
    return ()

  lax.fori_loop(0, KMAX, body, ())
  plsc.subcore_barrier()

  n0 = sid * NPS
  pltpu.sync_copy(acca_sh.at[pl.ds(n0, NPS)], acca.at[cid, pl.ds(n0, NPS)])
  pltpu.sync_copy(accb_sh.at[pl.ds(n0, NPS)], accb.at[cid, pl.ds(n0, NPS)])
  pltpu.sync_copy(accs_sh.at[pl.ds(n0, NPS)], accs.at[cid, pl.ds(n0, NPS)])


_sc_scatter = pl.kernel(
    _sc_scatter_body,
    compiler_params=_sc_params,
    out_type=(jax.ShapeDtypeStruct((NC, NPAD, HC), jnp.float32),
              jax.ShapeDtypeStruct((NC, NPAD, HC), jnp.float32),
              jax.ShapeDtypeStruct((NC, NPAD, HC), jnp.float32)),
    mesh=_sc_mesh,
    scratch_types=[
        pltpu.VMEM((RPS, ROW), jnp.int32),
        pltpu.VMEM((QE, HC), jnp.float32),
        pltpu.VMEM((QE, HC), jnp.float32),
        pltpu.VMEM((QE, HC), jnp.float32),
        pltpu.VMEM_SHARED((NPAD, HC), jnp.float32),
        pltpu.VMEM_SHARED((NPAD, HC), jnp.float32),
        pltpu.VMEM_SHARED((NPAD, HC), jnp.float32),
    ],
)


BE = 12800
B4 = BE // 4
BN = 2000


def _eproj_body(eat_ref, we1_ref, we2_ref, e1_ref, e2_ref):
  eat = eat_ref[...]
  e1_ref[...] = jax.lax.dot_general(
      eat, we1_ref[...], (((0,), (0,)), ((), ())),
      preferred_element_type=jnp.float32)
  e2_ref[...] = jax.lax.dot_general(
      eat, we2_ref[...], (((0,), (0,)), ((), ())),
      preferred_element_type=jnp.float32)


def _tc_eproj(eat, we1, we2):
  return pl.pallas_call(
      _eproj_body,
      grid=(E // BE,),
      in_specs=[
          pl.BlockSpec((EDGE_DIM, BE), lambda i: (0, i)),
          pl.BlockSpec((EDGE_DIM, HC), lambda i: (0, 0)),
          pl.BlockSpec((EDGE_DIM, HC), lambda i: (0, 0)),
      ],
      out_specs=[
          pl.BlockSpec((BE, HC), lambda i: (i, 0)),
          pl.BlockSpec((BE, HC), lambda i: (i, 0)),
      ],
      out_shape=[
          jax.ShapeDtypeStruct((E, HC), jnp.float32),
          jax.ShapeDtypeStruct((E, HC), jnp.float32),
      ],
  )(eat, we1, we2)


def _edgewise_body(selatt_ref, selexp_ref, gl_ref, gr_ref, e4_ref,
                   pa_ref, pb_ref):
  gl4 = gl_ref[...]
  e4 = e4_ref[...]
  m4 = gl4 + gr_ref[...] + e4
  m4 = jnp.maximum(m4, 0.2 * m4)
  alpha8 = jnp.dot(m4, selatt_ref[...], preferred_element_type=jnp.float32,
                   precision=jax.lax.Precision.HIGHEST)
  ex8 = jnp.exp(alpha8)
  exexp = jnp.dot(ex8, selexp_ref[...], preferred_element_type=jnp.float32,
                  precision=jax.lax.Precision.HIGHEST)
  pa_ref[...] = gl4 * exexp
  one = jnp.ones((B4, 1), jnp.float32)
  z = jnp.zeros((B4, 29), jnp.float32)
  pb_ref[...] = jnp.concatenate(
      [ex8[:, 0:2], one, z, ex8[:, 2:4], one, z,
       ex8[:, 4:6], one, z, ex8[:, 6:8], one, z], axis=1)


def _tc_edgewise(selatt, selexp, gl4, gr4, e4):
  return pl.pallas_call(
      _edgewise_body,
      grid=(E // BE,),
      in_specs=[
          pl.BlockSpec((128, 8), lambda i: (0, 0)),
          pl.BlockSpec((8, 128), lambda i: (0, 0)),
          pl.BlockSpec((B4, 128), lambda i: (i, 0)),
          pl.BlockSpec((B4, 128), lambda i: (i, 0)),
          pl.BlockSpec((B4, 128), lambda i: (i, 0)),
      ],
      out_specs=[
          pl.BlockSpec((B4, 128), lambda i: (i, 0)),
          pl.BlockSpec((B4, 128), lambda i: (i, 0)),
      ],
      out_shape=[
          jax.ShapeDtypeStruct((E // 4, 128), jnp.float32),
          jax.ShapeDtypeStruct((E // 4, 128), jnp.float32),
      ],
  )(selatt, selexp, gl4, gr4, e4)


def _xproj_body(x_ref, wl_ref, bl_ref, wr_ref, br_ref, xl_ref, xr_ref):
  x = x_ref[...]
  xl_ref[...] = jnp.dot(
      x, wl_ref[...], preferred_element_type=jnp.float32) + bl_ref[...]
  xr_ref[...] = jnp.dot(
      x, wr_ref[...], preferred_element_type=jnp.float32) + br_ref[...]


def _tc_xproj(x, wl, bl, wr, br):
  d = x.shape[1]
  return pl.pallas_call(
      _xproj_body,
      grid=(N // BN,),
      in_specs=[
          pl.BlockSpec((BN, d), lambda i: (i, 0)),
          pl.BlockSpec((d, HC), lambda i: (0, 0)),
          pl.BlockSpec((1, HC), lambda i: (0, 0)),
          pl.BlockSpec((d, HC), lambda i: (0, 0)),
          pl.BlockSpec((1, HC), lambda i: (0, 0)),
      ],
      out_specs=[
          pl.BlockSpec((BN, HC), lambda i: (i, 0)),
          pl.BlockSpec((BN, HC), lambda i: (i, 0)),
      ],
      out_shape=[
          jax.ShapeDtypeStruct((N, HC), jnp.float32),
          jax.ShapeDtypeStruct((N, HC), jnp.float32),
      ],
  )(x, wl, bl, wr, br)


def _lrelu(x):
  return jnp.maximum(x, 0.2 * x)


def _combine(acca_ref, accb_ref, accs_ref, xl, xr, attf, bias):
  num = acca_ref[0] + acca_ref[1]
  db = accb_ref[0] + accb_ref[1]
  s = accs_ref[0] + accs_ref[1]
  deg = jnp.maximum(db[:, 2:3], 1.0)
  e_loop = s / deg
  t = _lrelu(xl + xr + e_loop) * attf
  a0 = jnp.sum(t[:, :EMB], axis=1, keepdims=True)
  a1 = jnp.sum(t[:, EMB:], axis=1, keepdims=True)
  ex0 = jnp.exp(a0)
  ex1 = jnp.exp(a1)
  d0 = db[:, 0:1] + ex0 + 1e-16
  d1 = db[:, 1:2] + ex1 + 1e-16
  h0 = (num[:, :EMB] + xl[:, :EMB] * ex0) / d0
  h1 = (num[:, EMB:] + xl[:, EMB:] * ex1) / d1
  return jnp.concatenate([h0, h1], axis=1) + bias


def _node2_body(acca_ref, accb_ref, accs_ref, xl1_ref, xr1_ref, attf_ref,
                bias1_ref, wl_ref, bl_ref, wr_ref, br_ref,
                xl2_ref, xr2_ref):
  h = _combine(acca_ref, accb_ref, accs_ref, xl1_ref[...], xr1_ref[...],
               attf_ref[...], bias1_ref[...])
  xl2_ref[...] = jnp.dot(
      h, wl_ref[...], preferred_element_type=jnp.float32) + bl_ref[...]
  xr2_ref[...] = jnp.dot(
      h, wr_ref[...], preferred_element_type=jnp.float32) + br_ref[...]


def _acc_specs():
  return [
      pl.BlockSpec((NC, BN, HC), lambda i: (0, i, 0)),
      pl.BlockSpec((NC, BN, HC), lambda i: (0, i, 0)),
      pl.BlockSpec((NC, BN, HC), lambda i: (0, i, 0)),
  ]


def _tc_node2(acca, accb, accs, xl1, xr1, attf, bias1, wl, bl, wr, br):
  return pl.pallas_call(
      _node2_body,
      grid=(N // BN,),
      in_specs=_acc_specs() + [
          pl.BlockSpec((BN, HC), lambda i: (i, 0)),
          pl.BlockSpec((BN, HC), lambda i: (i, 0)),
          pl.BlockSpec((1, HC), lambda i: (0, 0)),
          pl.BlockSpec((1, HC), lambda i: (0, 0)),
          pl.BlockSpec((HC, HC), lambda i: (0, 0)),
          pl.BlockSpec((1, HC), lambda i: (0, 0)),
          pl.BlockSpec((HC, HC), lambda i: (0, 0)),
          pl.BlockSpec((1, HC), lambda i: (0, 0)),
      ],
      out_specs=[
          pl.BlockSpec((BN, HC), lambda i: (i, 0)),
          pl.BlockSpec((BN, HC), lambda i: (i, 0)),
      ],
      out_shape=[
          jax.ShapeDtypeStruct((N, HC), jnp.float32),
          jax.ShapeDtypeStruct((N, HC), jnp.float32),
      ],
  )(acca, accb, accs, xl1, xr1, attf, bias1, wl, bl, wr, br)


def _final_body(acca_ref, accb_ref, accs_ref, xl2_ref, xr2_ref, attf_ref,
                bias2_ref, wd1_ref, bd1_ref, wd2_ref, bd2_ref, q_ref):
  h = _combine(acca_ref, accb_ref, accs_ref, xl2_ref[...], xr2_ref[...],
               attf_ref[...], bias2_ref[...])
  t = jnp.maximum(
      jnp.dot(h, wd1_ref[...], preferred_element_type=jnp.float32)
      + bd1_ref[...], 0.0)
  q_ref[...] = jnp.dot(
      t, wd2_ref[...], preferred_element_type=jnp.float32) + bd2_ref[...]


def _tc_final(acca, accb, accs, xl2, xr2, attf, bias2, wd1, bd1, wd2, bd2):
  return pl.pallas_call(
      _final_body,
      grid=(N // BN,),
      in_specs=_acc_specs() + [
          pl.BlockSpec((BN, HC), lambda i: (i, 0)),
          pl.BlockSpec((BN, HC), lambda i: (i, 0)),
          pl.BlockSpec((1, HC), lambda i: (0, 0)),
          pl.BlockSpec((1, HC), lambda i: (0, 0)),
          pl.BlockSpec((HC, HID), lambda i: (0, 0)),
          pl.BlockSpec((1, HID), lambda i: (0, 0)),
          pl.BlockSpec((HID, OUT), lambda i: (0, 0)),
          pl.BlockSpec((1, OUT), lambda i: (0, 0)),
      ],
      out_specs=pl.BlockSpec((BN, OUT), lambda i: (i, 0)),
      out_shape=jax.ShapeDtypeStruct((N, OUT), jnp.float32),
  )(acca, accb, accs, xl2, xr2, attf, bias2, wd1, bd1, wd2, bd2)


def _selectors(att):
  attf = att.reshape(HC)
  cols = jnp.arange(128)
  grp = cols // 32
  comp = cols % 32
  head = comp // EMB
  sel = jnp.zeros((128, 8), jnp.float32).at[cols, 2 * grp + head].set(attf[comp])
  selexp = jnp.zeros((8, 128), jnp.float32).at[2 * grp + head, cols].set(1.0)
  return sel, selexp


def _gat_layer(src3, dst3, xl, xr, e_rows, selatt, selexp):
  gl, gr = _sc_gather(src3, dst3, xl, xr)
  gl4 = gl.reshape(E // 4, 128)
  gr4 = gr.reshape(E // 4, 128)
  e4 = e_rows.reshape(E // 4, 128)
  pa, pb = _tc_edgewise(selatt, selexp, gl4, gr4, e4)
  return _sc_scatter(dst3, pa.reshape(E, HC), pb.reshape(E, HC),
                     e4.reshape(E, HC), jnp.zeros((NPAD, HC), jnp.float32))


def kernel(x, edge_index, edge_attr, W_l1, b_l1, W_r1, b_r1, W_e1, att1,
           bias1, W_l2, b_l2, W_r2, b_r2, W_e2, att2, bias2, Wd1, bd1,
           Wd2, bd2):
  src3 = edge_index[0].reshape(NSUP, RPS, ROW)
  dst3 = edge_index[1].reshape(NSUP, RPS, ROW)
  attf1 = att1.reshape(1, HC)
  attf2 = att2.reshape(1, HC)
  selatt1, selexp = _selectors(att1)
  selatt2, _ = _selectors(att2)

  e1, e2 = _tc_eproj(edge_attr.T, W_e1, W_e2)
  xl1, xr1 = _tc_xproj(x, W_l1, b_l1.reshape(1, HC), W_r1, b_r1.reshape(1, HC))
  acca1, accb1, accs1 = _gat_layer(src3, dst3, xl1, xr1, e1, selatt1, selexp)
  xl2, xr2 = _tc_node2(acca1, accb1, accs1, xl1, xr1, attf1,
                       bias1.reshape(1, HC), W_l2, b_l2.reshape(1, HC),
                       W_r2, b_r2.reshape(1, HC))
  acca2, accb2, accs2 = _gat_layer(src3, dst3, xl2, xr2, e2, selatt2, selexp)
  q = _tc_final(acca2, accb2, accs2, xl2, xr2, attf2, bias2.reshape(1, HC),
                Wd1, bd1.reshape(1, HID), Wd2, bd2.reshape(1, OUT))
  return q
